# baseline (speedup 1.0000x reference)
import functools

import jax
import jax.numpy as jnp
from jax import lax
from jax.experimental import pallas as pl
from jax.experimental.pallas import tpu as pltpu
from jax.experimental.pallas import tpu_sc as plsc

_INPUT_LENGTH = 16384
_EMBED_DIM = 128
_REPEATS = 8
_TOTAL_LENGTH = 131072

_NUM_CORES = 2
_NUM_SUBCORES = 16
_NUM_WORKERS = _NUM_CORES * _NUM_SUBCORES
_ROWS_PER_WORKER = _INPUT_LENGTH // _NUM_WORKERS
_NUM_CHUNKS = 2
_CHUNK_ROWS = _ROWS_PER_WORKER // _NUM_CHUNKS


@functools.partial(
    pl.kernel,
    mesh=plsc.VectorSubcoreMesh(core_axis_name="c", subcore_axis_name="s"),
    out_type=jax.ShapeDtypeStruct((_TOTAL_LENGTH, _EMBED_DIM), jnp.float32),
    scratch_types=[
        pltpu.VMEM((_ROWS_PER_WORKER, _EMBED_DIM), jnp.float32),
        pltpu.SemaphoreType.DMA,
        pltpu.SemaphoreType.DMA,
        pltpu.SemaphoreType.DMA,
    ],
)
def _tile_kernel(x_hbm, out_hbm, buf, rd_sem0, rd_sem1, wr_sem):
    wid = lax.axis_index("s") * _NUM_CORES + lax.axis_index("c")
    base = wid * _ROWS_PER_WORKER

    rd_sems = [rd_sem0, rd_sem1]
    reads = []
    for c in range(_NUM_CHUNKS):
        src = x_hbm.at[pl.ds(base + c * _CHUNK_ROWS, _CHUNK_ROWS)]
        dst = buf.at[pl.ds(c * _CHUNK_ROWS, _CHUNK_ROWS)]
        reads.append(pltpu.make_async_copy(src, dst, rd_sems[c % 2]))
    reads[0].start()

    writes = []
    for c in range(_NUM_CHUNKS):
        if c + 1 < _NUM_CHUNKS:
            reads[c + 1].start()
        reads[c].wait()
        src = buf.at[pl.ds(c * _CHUNK_ROWS, _CHUNK_ROWS)]
        for r in range(_REPEATS):
            dst = out_hbm.at[
                pl.ds(r * _INPUT_LENGTH + base + c * _CHUNK_ROWS, _CHUNK_ROWS)
            ]
            writes.append(pltpu.make_async_copy(src, dst, wr_sem))
            writes[-1].start()
    for w in writes:
        w.wait()


def kernel(x):
    return _tile_kernel(x)


# speedup vs baseline: 1.0064x; 1.0064x over previous
import functools

import jax
import jax.numpy as jnp
from jax import lax
from jax.experimental import pallas as pl
from jax.experimental.pallas import tpu as pltpu
from jax.experimental.pallas import tpu_sc as plsc

_INPUT_LENGTH = 16384
_EMBED_DIM = 128
_REPEATS = 8
_TOTAL_LENGTH = 131072

_NUM_CORES = 2
_NUM_SUBCORES = 16
_NUM_WORKERS = _NUM_CORES * _NUM_SUBCORES
_ROWS_PER_WORKER = _INPUT_LENGTH // _NUM_WORKERS


@functools.partial(
    pl.kernel,
    mesh=plsc.VectorSubcoreMesh(core_axis_name="c", subcore_axis_name="s"),
    out_type=jax.ShapeDtypeStruct((_TOTAL_LENGTH, _EMBED_DIM), jnp.float32),
    scratch_types=[
        pltpu.VMEM((_ROWS_PER_WORKER, _EMBED_DIM), jnp.float32),
        pltpu.SemaphoreType.DMA,
    ],
)
def _tile_kernel(x_hbm, out_hbm, buf, sem):
    wid = lax.axis_index("s") * _NUM_CORES + lax.axis_index("c")
    base = wid * _ROWS_PER_WORKER
    pltpu.sync_copy(x_hbm.at[pl.ds(base, _ROWS_PER_WORKER)], buf)
    copies = []
    for r in range(_REPEATS):
        dst = out_hbm.at[pl.ds(r * _INPUT_LENGTH + base, _ROWS_PER_WORKER)]
        copies.append(pltpu.make_async_copy(buf, dst, sem))
        copies[-1].start()
    for c in copies:
        c.wait()


def kernel(x):
    return _tile_kernel(x)
